# Initial kernel scaffold; baseline (speedup 1.0000x reference)
import jax
import jax.numpy as jnp
from jax.experimental import pallas as pl


def kernel(edge_features, node_features, senders, receivers, W_edge, W_sender, W_receiver, bias):
    raise NotImplementedError("write your pallas kernel here")


# speedup vs baseline: 2.0854x; 2.0854x over previous
import functools

import jax
import jax.numpy as jnp
from jax import lax
from jax.experimental import pallas as pl
from jax.experimental.pallas import tpu as pltpu
from jax.experimental.pallas import tpu_sc as plsc

_WIN = 128
_LANES = 16

_NODE_BLK = 2000
_EDGE_BLK = 8000


def _proj_body(nf_ref, ws_ref, wr_ref, b_ref, sp_ref, rp_ref):
    x = nf_ref[...]
    dn = (((1,), (1,)), ((), ()))
    sp_ref[...] = lax.dot_general(x, ws_ref[...], dn,
                                  preferred_element_type=jnp.float32) + b_ref[...]
    rp_ref[...] = lax.dot_general(x, wr_ref[...], dn,
                                  preferred_element_type=jnp.float32)


def _node_proj(node_features, w_s, w_r, bias2d):
    n_nodes, d_feat = node_features.shape
    out_dim = w_s.shape[0]
    blk = _NODE_BLK if n_nodes % _NODE_BLK == 0 else n_nodes
    grid = (n_nodes // blk,)
    return pl.pallas_call(
        _proj_body,
        grid=grid,
        in_specs=[
            pl.BlockSpec((blk, d_feat), lambda i: (i, 0)),
            pl.BlockSpec((out_dim, d_feat), lambda i: (0, 0)),
            pl.BlockSpec((out_dim, d_feat), lambda i: (0, 0)),
            pl.BlockSpec((1, out_dim), lambda i: (0, 0)),
        ],
        out_specs=[
            pl.BlockSpec((blk, out_dim), lambda i: (i, 0)),
            pl.BlockSpec((blk, out_dim), lambda i: (i, 0)),
        ],
        out_shape=[
            jax.ShapeDtypeStruct((n_nodes, out_dim), jnp.float32),
            jax.ShapeDtypeStruct((n_nodes, out_dim), jnp.float32),
        ],
    )(node_features, w_s, w_r, bias2d)


def _edge_body(ef_ref, we_ref, g_ref, o_ref):
    dn = (((1,), (1,)), ((), ()))
    o_ref[...] = lax.dot_general(ef_ref[...], we_ref[...], dn,
                                 preferred_element_type=jnp.float32) + g_ref[...]


def _edge_combine(edge_features, w_e, gathered):
    n_edges, d_edge = edge_features.shape
    out_dim = w_e.shape[0]
    blk = _EDGE_BLK if n_edges % _EDGE_BLK == 0 else n_edges
    grid = (n_edges // blk,)
    return pl.pallas_call(
        _edge_body,
        grid=grid,
        in_specs=[
            pl.BlockSpec((blk, d_edge), lambda i: (i, 0)),
            pl.BlockSpec((out_dim, d_edge), lambda i: (0, 0)),
            pl.BlockSpec((blk, out_dim), lambda i: (i, 0)),
        ],
        out_specs=pl.BlockSpec((blk, out_dim), lambda i: (i, 0)),
        out_shape=jax.ShapeDtypeStruct((n_edges, out_dim), jnp.float32),
    )(edge_features, w_e, gathered)


def _gather_add(sp, rp, senders2d, receivers2d):
    n_edges = senders2d.shape[1]
    out_dim = sp.shape[1]
    mesh = plsc.VectorSubcoreMesh(core_axis_name="core", subcore_axis_name="subcore")

    @functools.partial(
        pl.kernel,
        out_type=jax.ShapeDtypeStruct((n_edges, out_dim), jnp.float32),
        mesh=mesh,
        scratch_types=[
            pltpu.VMEM((_WIN, out_dim), jnp.float32),
            pltpu.VMEM((_WIN, out_dim), jnp.float32),
            pltpu.SemaphoreType.DMA,
            pltpu.SemaphoreType.DMA,
        ],
    )
    def sc_kernel(sp_hbm, rp_hbm, s_hbm, r_hbm, o_hbm, g1, g2, sem1, sem2):
        def body(i_s, i_r, o_vmem):
            c1 = pltpu.async_copy(sp_hbm.at[i_s.at[0]], g1, sem1)
            c2 = pltpu.async_copy(rp_hbm.at[i_r.at[0]], g2, sem2)
            c1.wait()
            c2.wait()

            @pl.loop(0, _WIN)
            def _(i):
                for j in range(out_dim // _LANES):
                    sl = pl.ds(j * _LANES, _LANES)
                    o_vmem[i, sl] = g1[i, sl] + g2[i, sl]

        pltpu.emit_pipeline(
            body,
            grid=(n_edges // _WIN,),
            in_specs=[
                pl.BlockSpec((1, _WIN), lambda i: (0, i)),
                pl.BlockSpec((1, _WIN), lambda i: (0, i)),
            ],
            out_specs=[pl.BlockSpec((_WIN, out_dim), lambda i: (i, 0))],
            core_axis_name=("core", "subcore"),
            dimension_semantics=(pltpu.PARALLEL,),
        )(s_hbm, r_hbm, o_hbm)

    return sc_kernel(sp, rp, senders2d, receivers2d)


def kernel(edge_features, node_features, senders, receivers,
           W_edge, W_sender, W_receiver, bias):
    senders2d = senders.astype(jnp.int32).reshape(1, -1)
    receivers2d = receivers.astype(jnp.int32).reshape(1, -1)
    sp, rp = _node_proj(node_features, W_sender, W_receiver, bias.reshape(1, -1))
    gathered = _gather_add(sp, rp, senders2d, receivers2d)
    return _edge_combine(edge_features, W_edge, gathered)


# speedup vs baseline: 3.4230x; 1.6414x over previous
import functools

import jax
import jax.numpy as jnp
from jax import lax
from jax.experimental import pallas as pl
from jax.experimental.pallas import tpu as pltpu
from jax.experimental.pallas import tpu_sc as plsc

_WIN = 128
_LANES = 16

_NODE_BLK = 2000
_EDGE_BLK = 8000


def _proj_body(nf_ref, ws_ref, wr_ref, b_ref, sp_ref, rp_ref):
    x = nf_ref[...]
    dn = (((1,), (1,)), ((), ()))
    sp_ref[...] = lax.dot_general(x, ws_ref[...], dn,
                                  preferred_element_type=jnp.float32) + b_ref[...]
    rp_ref[...] = lax.dot_general(x, wr_ref[...], dn,
                                  preferred_element_type=jnp.float32)


def _node_proj(node_features, w_s, w_r, bias2d):
    n_nodes, d_feat = node_features.shape
    out_dim = w_s.shape[0]
    blk = _NODE_BLK if n_nodes % _NODE_BLK == 0 else n_nodes
    grid = (n_nodes // blk,)
    return pl.pallas_call(
        _proj_body,
        grid=grid,
        in_specs=[
            pl.BlockSpec((blk, d_feat), lambda i: (i, 0)),
            pl.BlockSpec((out_dim, d_feat), lambda i: (0, 0)),
            pl.BlockSpec((out_dim, d_feat), lambda i: (0, 0)),
            pl.BlockSpec((1, out_dim), lambda i: (0, 0)),
        ],
        out_specs=[
            pl.BlockSpec((blk, out_dim), lambda i: (i, 0)),
            pl.BlockSpec((blk, out_dim), lambda i: (i, 0)),
        ],
        out_shape=[
            jax.ShapeDtypeStruct((n_nodes, out_dim), jnp.float32),
            jax.ShapeDtypeStruct((n_nodes, out_dim), jnp.float32),
        ],
    )(node_features, w_s, w_r, bias2d)


def _edge_body(ef_ref, we_ref, g_ref, o_ref):
    dn = (((1,), (1,)), ((), ()))
    o_ref[...] = lax.dot_general(ef_ref[...], we_ref[...], dn,
                                 preferred_element_type=jnp.float32) + g_ref[...]


def _edge_combine(edge_features, w_e, gathered):
    n_edges, d_edge = edge_features.shape
    out_dim = w_e.shape[0]
    blk = _EDGE_BLK if n_edges % _EDGE_BLK == 0 else n_edges
    grid = (n_edges // blk,)
    return pl.pallas_call(
        _edge_body,
        grid=grid,
        in_specs=[
            pl.BlockSpec((blk, d_edge), lambda i: (i, 0)),
            pl.BlockSpec((out_dim, d_edge), lambda i: (0, 0)),
            pl.BlockSpec((blk, out_dim), lambda i: (i, 0)),
        ],
        out_specs=pl.BlockSpec((blk, out_dim), lambda i: (i, 0)),
        out_shape=jax.ShapeDtypeStruct((n_edges, out_dim), jnp.float32),
    )(edge_features, w_e, gathered)


def _gather_add(sp, rp, senders2d, receivers2d):
    n_edges = senders2d.shape[1]
    out_dim = sp.shape[1]
    mesh = plsc.VectorSubcoreMesh(core_axis_name="core", subcore_axis_name="subcore")

    @functools.partial(
        pl.kernel,
        out_type=jax.ShapeDtypeStruct((n_edges, out_dim), jnp.float32),
        mesh=mesh,
        scratch_types=[
            pltpu.VMEM((_WIN, out_dim), jnp.float32),
            pltpu.VMEM((_WIN, out_dim), jnp.float32),
            pltpu.SemaphoreType.DMA,
            pltpu.SemaphoreType.DMA,
        ],
    )
    def sc_kernel(sp_hbm, rp_hbm, s_hbm, r_hbm, o_hbm, g1, g2, sem1, sem2):
        def body(i_s, i_r, o_vmem):
            c1 = pltpu.async_copy(sp_hbm.at[i_s.at[0]], g1, sem1)
            c2 = pltpu.async_copy(rp_hbm.at[i_r.at[0]], g2, sem2)
            c1.wait()
            c2.wait()

            @plsc.parallel_loop(0, _WIN, unroll=2)
            def _(i):
                for j in range(out_dim // _LANES):
                    sl = pl.ds(j * _LANES, _LANES)
                    o_vmem[i, sl] = g1[i, sl] + g2[i, sl]

        pltpu.emit_pipeline(
            body,
            grid=(n_edges // _WIN,),
            in_specs=[
                pl.BlockSpec((1, _WIN), lambda i: (0, i)),
                pl.BlockSpec((1, _WIN), lambda i: (0, i)),
            ],
            out_specs=[pl.BlockSpec((_WIN, out_dim), lambda i: (i, 0))],
            core_axis_name=("core", "subcore"),
            dimension_semantics=(pltpu.PARALLEL,),
        )(s_hbm, r_hbm, o_hbm)

    return sc_kernel(sp, rp, senders2d, receivers2d)


def kernel(edge_features, node_features, senders, receivers,
           W_edge, W_sender, W_receiver, bias):
    senders2d = senders.astype(jnp.int32).reshape(1, -1)
    receivers2d = receivers.astype(jnp.int32).reshape(1, -1)
    sp, rp = _node_proj(node_features, W_sender, W_receiver, bias.reshape(1, -1))
    gathered = _gather_add(sp, rp, senders2d, receivers2d)
    return _edge_combine(edge_features, W_edge, gathered)


# speedup vs baseline: 3.4338x; 1.0032x over previous
import functools

import jax
import jax.numpy as jnp
from jax import lax
from jax.experimental import pallas as pl
from jax.experimental.pallas import tpu as pltpu
from jax.experimental.pallas import tpu_sc as plsc

_WIN = 128
_LANES = 16

_NODE_BLK = 2000
_EDGE_BLK = 8000


def _proj_body(nf_ref, ws_ref, wr_ref, b_ref, sp_ref, rp_ref):
    x = nf_ref[...]
    dn = (((1,), (1,)), ((), ()))
    sp_ref[...] = lax.dot_general(x, ws_ref[...], dn,
                                  preferred_element_type=jnp.float32) + b_ref[...]
    rp_ref[...] = lax.dot_general(x, wr_ref[...], dn,
                                  preferred_element_type=jnp.float32)


def _node_proj(node_features, w_s, w_r, bias2d):
    n_nodes, d_feat = node_features.shape
    out_dim = w_s.shape[0]
    blk = _NODE_BLK if n_nodes % _NODE_BLK == 0 else n_nodes
    grid = (n_nodes // blk,)
    return pl.pallas_call(
        _proj_body,
        grid=grid,
        in_specs=[
            pl.BlockSpec((blk, d_feat), lambda i: (i, 0)),
            pl.BlockSpec((out_dim, d_feat), lambda i: (0, 0)),
            pl.BlockSpec((out_dim, d_feat), lambda i: (0, 0)),
            pl.BlockSpec((1, out_dim), lambda i: (0, 0)),
        ],
        out_specs=[
            pl.BlockSpec((blk, out_dim), lambda i: (i, 0)),
            pl.BlockSpec((blk, out_dim), lambda i: (i, 0)),
        ],
        out_shape=[
            jax.ShapeDtypeStruct((n_nodes, out_dim), jnp.float32),
            jax.ShapeDtypeStruct((n_nodes, out_dim), jnp.float32),
        ],
    )(node_features, w_s, w_r, bias2d)


def _edge_body(ef_ref, we_ref, g_ref, o_ref):
    dn = (((1,), (1,)), ((), ()))
    o_ref[...] = lax.dot_general(ef_ref[...], we_ref[...], dn,
                                 preferred_element_type=jnp.float32) + g_ref[...]


def _edge_combine(edge_features, w_e, gathered):
    n_edges, d_edge = edge_features.shape
    out_dim = w_e.shape[0]
    blk = _EDGE_BLK if n_edges % _EDGE_BLK == 0 else n_edges
    grid = (n_edges // blk,)
    return pl.pallas_call(
        _edge_body,
        grid=grid,
        in_specs=[
            pl.BlockSpec((blk, d_edge), lambda i: (i, 0)),
            pl.BlockSpec((out_dim, d_edge), lambda i: (0, 0)),
            pl.BlockSpec((blk, out_dim), lambda i: (i, 0)),
        ],
        out_specs=pl.BlockSpec((blk, out_dim), lambda i: (i, 0)),
        out_shape=jax.ShapeDtypeStruct((n_edges, out_dim), jnp.float32),
    )(edge_features, w_e, gathered)


def _gather_add(sp, rp, senders2d, receivers2d):
    n_edges = senders2d.shape[1]
    out_dim = sp.shape[1]
    mesh = plsc.VectorSubcoreMesh(core_axis_name="core", subcore_axis_name="subcore")

    @functools.partial(
        pl.kernel,
        out_type=jax.ShapeDtypeStruct((n_edges, out_dim), jnp.float32),
        mesh=mesh,
        scratch_types=[
            pltpu.VMEM((_WIN, out_dim), jnp.float32),
            pltpu.VMEM((_WIN, out_dim), jnp.float32),
            pltpu.SemaphoreType.DMA,
            pltpu.SemaphoreType.DMA,
        ],
    )
    def sc_kernel(sp_hbm, rp_hbm, s_hbm, r_hbm, o_hbm, g1, g2, sem1, sem2):
        def body(i_s, i_r, o_vmem):
            c1 = pltpu.async_copy(sp_hbm.at[i_s.at[0]], g1, sem1)
            c2 = pltpu.async_copy(rp_hbm.at[i_r.at[0]], g2, sem2)
            c1.wait()
            c2.wait()

            @plsc.parallel_loop(0, _WIN, unroll=4)
            def _(i):
                for j in range(out_dim // _LANES):
                    sl = pl.ds(j * _LANES, _LANES)
                    o_vmem[i, sl] = g1[i, sl] + g2[i, sl]

        pltpu.emit_pipeline(
            body,
            grid=(n_edges // _WIN,),
            in_specs=[
                pl.BlockSpec((1, _WIN), lambda i: (0, i)),
                pl.BlockSpec((1, _WIN), lambda i: (0, i)),
            ],
            out_specs=[pl.BlockSpec((_WIN, out_dim), lambda i: (i, 0))],
            core_axis_name=("core", "subcore"),
            dimension_semantics=(pltpu.PARALLEL,),
        )(s_hbm, r_hbm, o_hbm)

    return sc_kernel(sp, rp, senders2d, receivers2d)


def kernel(edge_features, node_features, senders, receivers,
           W_edge, W_sender, W_receiver, bias):
    senders2d = senders.astype(jnp.int32).reshape(1, -1)
    receivers2d = receivers.astype(jnp.int32).reshape(1, -1)
    sp, rp = _node_proj(node_features, W_sender, W_receiver, bias.reshape(1, -1))
    gathered = _gather_add(sp, rp, senders2d, receivers2d)
    return _edge_combine(edge_features, W_edge, gathered)


# speedup vs baseline: 3.4394x; 1.0016x over previous
import functools

import jax
import jax.numpy as jnp
from jax import lax
from jax.experimental import pallas as pl
from jax.experimental.pallas import tpu as pltpu
from jax.experimental.pallas import tpu_sc as plsc

_WIN = 128
_LANES = 16

_NODE_BLK = 2000
_EDGE_BLK = 8000


def _proj_body(nf_ref, ws_ref, wr_ref, b_ref, sp_ref, rp_ref):
    x = nf_ref[...]
    dn = (((1,), (1,)), ((), ()))
    sp_ref[...] = lax.dot_general(x, ws_ref[...], dn,
                                  preferred_element_type=jnp.float32) + b_ref[...]
    rp_ref[...] = lax.dot_general(x, wr_ref[...], dn,
                                  preferred_element_type=jnp.float32)


def _node_proj(node_features, w_s, w_r, bias2d):
    n_nodes, d_feat = node_features.shape
    out_dim = w_s.shape[0]
    blk = _NODE_BLK if n_nodes % _NODE_BLK == 0 else n_nodes
    grid = (n_nodes // blk,)
    return pl.pallas_call(
        _proj_body,
        grid=grid,
        in_specs=[
            pl.BlockSpec((blk, d_feat), lambda i: (i, 0)),
            pl.BlockSpec((out_dim, d_feat), lambda i: (0, 0)),
            pl.BlockSpec((out_dim, d_feat), lambda i: (0, 0)),
            pl.BlockSpec((1, out_dim), lambda i: (0, 0)),
        ],
        out_specs=[
            pl.BlockSpec((blk, out_dim), lambda i: (i, 0)),
            pl.BlockSpec((blk, out_dim), lambda i: (i, 0)),
        ],
        out_shape=[
            jax.ShapeDtypeStruct((n_nodes, out_dim), jnp.float32),
            jax.ShapeDtypeStruct((n_nodes, out_dim), jnp.float32),
        ],
    )(node_features, w_s, w_r, bias2d)


def _edge_body(ef_ref, we_ref, g_ref, o_ref):
    dn = (((1,), (1,)), ((), ()))
    o_ref[...] = lax.dot_general(ef_ref[...], we_ref[...], dn,
                                 preferred_element_type=jnp.float32) + g_ref[...]


def _edge_combine(edge_features, w_e, gathered):
    n_edges, d_edge = edge_features.shape
    out_dim = w_e.shape[0]
    blk = _EDGE_BLK if n_edges % _EDGE_BLK == 0 else n_edges
    grid = (n_edges // blk,)
    return pl.pallas_call(
        _edge_body,
        grid=grid,
        in_specs=[
            pl.BlockSpec((blk, d_edge), lambda i: (i, 0)),
            pl.BlockSpec((out_dim, d_edge), lambda i: (0, 0)),
            pl.BlockSpec((blk, out_dim), lambda i: (i, 0)),
        ],
        out_specs=pl.BlockSpec((blk, out_dim), lambda i: (i, 0)),
        out_shape=jax.ShapeDtypeStruct((n_edges, out_dim), jnp.float32),
    )(edge_features, w_e, gathered)


def _gather_add(sp, rp, senders2d, receivers2d):
    n_edges = senders2d.shape[1]
    out_dim = sp.shape[1]
    mesh = plsc.VectorSubcoreMesh(core_axis_name="core", subcore_axis_name="subcore")

    @functools.partial(
        pl.kernel,
        out_type=jax.ShapeDtypeStruct((n_edges, out_dim), jnp.float32),
        mesh=mesh,
        scratch_types=[
            pltpu.VMEM((_WIN, out_dim), jnp.float32),
            pltpu.SemaphoreType.DMA,
            pltpu.SemaphoreType.DMA,
        ],
    )
    def sc_kernel(sp_hbm, rp_hbm, s_hbm, r_hbm, o_hbm, g2, sem1, sem2):
        def body(i_s, i_r, o_vmem):
            c1 = pltpu.async_copy(sp_hbm.at[i_s.at[0]], o_vmem, sem1)
            c2 = pltpu.async_copy(rp_hbm.at[i_r.at[0]], g2, sem2)
            c1.wait()
            c2.wait()

            @plsc.parallel_loop(0, _WIN, unroll=4)
            def _(i):
                for j in range(out_dim // _LANES):
                    sl = pl.ds(j * _LANES, _LANES)
                    plsc.addupdate(o_vmem.at[i, sl], g2[i, sl])

        pltpu.emit_pipeline(
            body,
            grid=(n_edges // _WIN,),
            in_specs=[
                pl.BlockSpec((1, _WIN), lambda i: (0, i)),
                pl.BlockSpec((1, _WIN), lambda i: (0, i)),
            ],
            out_specs=[pl.BlockSpec((_WIN, out_dim), lambda i: (i, 0))],
            core_axis_name=("core", "subcore"),
            dimension_semantics=(pltpu.PARALLEL,),
        )(s_hbm, r_hbm, o_hbm)

    return sc_kernel(sp, rp, senders2d, receivers2d)


def kernel(edge_features, node_features, senders, receivers,
           W_edge, W_sender, W_receiver, bias):
    senders2d = senders.astype(jnp.int32).reshape(1, -1)
    receivers2d = receivers.astype(jnp.int32).reshape(1, -1)
    sp, rp = _node_proj(node_features, W_sender, W_receiver, bias.reshape(1, -1))
    gathered = _gather_add(sp, rp, senders2d, receivers2d)
    return _edge_combine(edge_features, W_edge, gathered)


# speedup vs baseline: 3.6196x; 1.0524x over previous
import functools

import jax
import jax.numpy as jnp
from jax import lax
from jax.experimental import pallas as pl
from jax.experimental.pallas import tpu as pltpu
from jax.experimental.pallas import tpu_sc as plsc

_WIN = 128
_LANES = 16

_NODE_BLK = 2000
_EDGE_BLK = 8000


def _proj_body(nf_ref, ws_ref, wr_ref, b_ref, sp_ref, rp_ref):
    x = nf_ref[...]
    dn = (((1,), (1,)), ((), ()))
    sp_ref[...] = lax.dot_general(x, ws_ref[...], dn,
                                  preferred_element_type=jnp.float32) + b_ref[...]
    rp_ref[...] = lax.dot_general(x, wr_ref[...], dn,
                                  preferred_element_type=jnp.float32)


def _node_proj(node_features, w_s, w_r, bias2d):
    n_nodes, d_feat = node_features.shape
    out_dim = w_s.shape[0]
    blk = _NODE_BLK if n_nodes % _NODE_BLK == 0 else n_nodes
    grid = (n_nodes // blk,)
    return pl.pallas_call(
        _proj_body,
        grid=grid,
        in_specs=[
            pl.BlockSpec((blk, d_feat), lambda i: (i, 0)),
            pl.BlockSpec((out_dim, d_feat), lambda i: (0, 0)),
            pl.BlockSpec((out_dim, d_feat), lambda i: (0, 0)),
            pl.BlockSpec((1, out_dim), lambda i: (0, 0)),
        ],
        out_specs=[
            pl.BlockSpec((blk, out_dim), lambda i: (i, 0)),
            pl.BlockSpec((blk, out_dim), lambda i: (i, 0)),
        ],
        out_shape=[
            jax.ShapeDtypeStruct((n_nodes, out_dim), jnp.float32),
            jax.ShapeDtypeStruct((n_nodes, out_dim), jnp.float32),
        ],
    )(node_features, w_s, w_r, bias2d)


def _edge_body(ef_ref, we_ref, g_ref, o_ref):
    dn = (((1,), (1,)), ((), ()))
    o_ref[...] = lax.dot_general(ef_ref[...], we_ref[...], dn,
                                 preferred_element_type=jnp.float32) + g_ref[...]


def _edge_body_alias(prev_ref, ef_ref, we_ref, g_ref, o_ref):
    del prev_ref
    _edge_body(ef_ref, we_ref, g_ref, o_ref)


def _edge_combine_chunk(out_prev, ef_c, w_e, g_c, chunk_idx, n_edges):
    sz, d_edge = ef_c.shape
    out_dim = w_e.shape[0]
    blk = _EDGE_BLK if sz % _EDGE_BLK == 0 else sz
    nblk = sz // blk
    base = chunk_idx * nblk
    specs = [
        pl.BlockSpec((blk, d_edge), lambda i: (i, 0)),
        pl.BlockSpec((out_dim, d_edge), lambda i: (0, 0)),
        pl.BlockSpec((blk, out_dim), lambda i: (i, 0)),
    ]
    out_spec = pl.BlockSpec((blk, out_dim), lambda i: (i + base, 0))
    out_shape = jax.ShapeDtypeStruct((n_edges, out_dim), jnp.float32)
    if out_prev is None:
        return pl.pallas_call(
            _edge_body,
            grid=(nblk,),
            in_specs=specs,
            out_specs=out_spec,
            out_shape=out_shape,
        )(ef_c, w_e, g_c)
    return pl.pallas_call(
        _edge_body_alias,
        grid=(nblk,),
        in_specs=[pl.BlockSpec(memory_space=pl.ANY)] + specs,
        out_specs=out_spec,
        out_shape=out_shape,
        input_output_aliases={0: 0},
    )(out_prev, ef_c, w_e, g_c)


def _gather_add(sp, rp, senders2d, receivers2d):
    n_edges = senders2d.shape[1]
    out_dim = sp.shape[1]
    mesh = plsc.VectorSubcoreMesh(core_axis_name="core", subcore_axis_name="subcore")

    @functools.partial(
        pl.kernel,
        out_type=jax.ShapeDtypeStruct((n_edges, out_dim), jnp.float32),
        mesh=mesh,
        scratch_types=[
            pltpu.VMEM((_WIN, out_dim), jnp.float32),
            pltpu.SemaphoreType.DMA,
            pltpu.SemaphoreType.DMA,
        ],
    )
    def sc_kernel(sp_hbm, rp_hbm, s_hbm, r_hbm, o_hbm, g2, sem1, sem2):
        def body(i_s, i_r, o_vmem):
            c1 = pltpu.async_copy(sp_hbm.at[i_s.at[0]], o_vmem, sem1)
            c2 = pltpu.async_copy(rp_hbm.at[i_r.at[0]], g2, sem2)
            c1.wait()
            c2.wait()

            @plsc.parallel_loop(0, _WIN, unroll=4)
            def _(i):
                for j in range(out_dim // _LANES):
                    sl = pl.ds(j * _LANES, _LANES)
                    plsc.addupdate(o_vmem.at[i, sl], g2[i, sl])

        pltpu.emit_pipeline(
            body,
            grid=(n_edges // _WIN,),
            in_specs=[
                pl.BlockSpec((1, _WIN), lambda i: (0, i)),
                pl.BlockSpec((1, _WIN), lambda i: (0, i)),
            ],
            out_specs=[pl.BlockSpec((_WIN, out_dim), lambda i: (i, 0))],
            core_axis_name=("core", "subcore"),
            dimension_semantics=(pltpu.PARALLEL,),
        )(s_hbm, r_hbm, o_hbm)

    return sc_kernel(sp, rp, senders2d, receivers2d)


_N_CHUNKS = 4


def kernel(edge_features, node_features, senders, receivers,
           W_edge, W_sender, W_receiver, bias):
    n_edges = edge_features.shape[0]
    senders2d = senders.astype(jnp.int32).reshape(1, -1)
    receivers2d = receivers.astype(jnp.int32).reshape(1, -1)
    sp, rp = _node_proj(node_features, W_sender, W_receiver, bias.reshape(1, -1))
    sz = n_edges // _N_CHUNKS
    out = None
    for c in range(_N_CHUNKS):
        sl = slice(c * sz, (c + 1) * sz)
        g_c = _gather_add(sp, rp, senders2d[:, sl], receivers2d[:, sl])
        out = _edge_combine_chunk(out, edge_features[sl], W_edge, g_c, c, n_edges)
    return out


# speedup vs baseline: 3.6381x; 1.0051x over previous
import functools

import jax
import jax.numpy as jnp
from jax import lax
from jax.experimental import pallas as pl
from jax.experimental.pallas import tpu as pltpu
from jax.experimental.pallas import tpu_sc as plsc

_WIN = 128
_LANES = 16

_NODE_BLK = 2000
_EDGE_BLK = 8000


def _proj_body(nf_ref, ws_ref, wr_ref, b_ref, sp_ref, rp_ref):
    x = nf_ref[...]
    dn = (((1,), (1,)), ((), ()))
    sp_ref[...] = lax.dot_general(x, ws_ref[...], dn,
                                  preferred_element_type=jnp.float32) + b_ref[...]
    rp_ref[...] = lax.dot_general(x, wr_ref[...], dn,
                                  preferred_element_type=jnp.float32)


def _node_proj(node_features, w_s, w_r, bias2d):
    n_nodes, d_feat = node_features.shape
    out_dim = w_s.shape[0]
    blk = _NODE_BLK if n_nodes % _NODE_BLK == 0 else n_nodes
    grid = (n_nodes // blk,)
    return pl.pallas_call(
        _proj_body,
        grid=grid,
        in_specs=[
            pl.BlockSpec((blk, d_feat), lambda i: (i, 0)),
            pl.BlockSpec((out_dim, d_feat), lambda i: (0, 0)),
            pl.BlockSpec((out_dim, d_feat), lambda i: (0, 0)),
            pl.BlockSpec((1, out_dim), lambda i: (0, 0)),
        ],
        out_specs=[
            pl.BlockSpec((blk, out_dim), lambda i: (i, 0)),
            pl.BlockSpec((blk, out_dim), lambda i: (i, 0)),
        ],
        out_shape=[
            jax.ShapeDtypeStruct((n_nodes, out_dim), jnp.float32),
            jax.ShapeDtypeStruct((n_nodes, out_dim), jnp.float32),
        ],
    )(node_features, w_s, w_r, bias2d)


def _edge_body(ef_ref, we_ref, g_ref, o_ref):
    dn = (((1,), (1,)), ((), ()))
    o_ref[...] = lax.dot_general(ef_ref[...], we_ref[...], dn,
                                 preferred_element_type=jnp.float32) + g_ref[...]


def _edge_body_alias(prev_ref, ef_ref, we_ref, g_ref, o_ref):
    del prev_ref
    _edge_body(ef_ref, we_ref, g_ref, o_ref)


def _edge_combine_chunk(out_prev, ef_c, w_e, g_c, chunk_idx, n_edges):
    sz, d_edge = ef_c.shape
    out_dim = w_e.shape[0]
    blk = _EDGE_BLK if sz % _EDGE_BLK == 0 else sz
    nblk = sz // blk
    base = chunk_idx * nblk
    specs = [
        pl.BlockSpec((blk, d_edge), lambda i: (i, 0)),
        pl.BlockSpec((out_dim, d_edge), lambda i: (0, 0)),
        pl.BlockSpec((blk, out_dim), lambda i: (i, 0)),
    ]
    out_spec = pl.BlockSpec((blk, out_dim), lambda i: (i + base, 0))
    out_shape = jax.ShapeDtypeStruct((n_edges, out_dim), jnp.float32)
    if out_prev is None:
        return pl.pallas_call(
            _edge_body,
            grid=(nblk,),
            in_specs=specs,
            out_specs=out_spec,
            out_shape=out_shape,
        )(ef_c, w_e, g_c)
    return pl.pallas_call(
        _edge_body_alias,
        grid=(nblk,),
        in_specs=[pl.BlockSpec(memory_space=pl.ANY)] + specs,
        out_specs=out_spec,
        out_shape=out_shape,
        input_output_aliases={0: 0},
    )(out_prev, ef_c, w_e, g_c)


_N_SUBCORES = 32


def _gather_add(sp, rp, senders2d, receivers2d):
    n_edges = senders2d.shape[1]
    out_dim = sp.shape[1]
    nwin = -(-n_edges // (_N_SUBCORES * _WIN))
    per_sc = nwin * _WIN
    mesh = plsc.VectorSubcoreMesh(core_axis_name="core", subcore_axis_name="subcore")

    @functools.partial(
        pl.kernel,
        out_type=jax.ShapeDtypeStruct((n_edges, out_dim), jnp.float32),
        mesh=mesh,
        scratch_types=[
            pltpu.VMEM((per_sc,), jnp.int32),
            pltpu.VMEM((per_sc,), jnp.int32),
            pltpu.VMEM((_WIN, out_dim), jnp.float32),
            pltpu.VMEM((_WIN, out_dim), jnp.float32),
            pltpu.VMEM((_WIN, out_dim), jnp.float32),
            pltpu.VMEM((_WIN, out_dim), jnp.float32),
        ] + [pltpu.SemaphoreType.DMA] * 8,
    )
    def sc_kernel(sp_hbm, rp_hbm, s_hbm, r_hbm, o_hbm,
                  sidx, ridx, spb0, spb1, rpb0, rpb1,
                  sem_si, sem_ri, s_s0, s_s1, s_r0, s_r1, s_o0, s_o1):
        core = lax.axis_index("core")
        sub = lax.axis_index("subcore")
        gid = core * (_N_SUBCORES // 2) + sub
        e0 = jnp.minimum(gid * per_sc, n_edges - per_sc)
        ci = pltpu.async_copy(s_hbm.at[0, pl.ds(e0, per_sc)], sidx, sem_si)
        cr = pltpu.async_copy(r_hbm.at[0, pl.ds(e0, per_sc)], ridx, sem_ri)
        ci.wait()
        cr.wait()
        spb = [spb0, spb1]
        rpb = [rpb0, rpb1]
        ssem = [s_s0, s_s1]
        rsem = [s_r0, s_r1]
        osem = [s_o0, s_o1]
        g_hand = [None, None]
        o_hand = [None, None]

        def add_window(sb, rb):
            @plsc.parallel_loop(0, _WIN, unroll=4)
            def _(i):
                for j in range(out_dim // _LANES):
                    sl = pl.ds(j * _LANES, _LANES)
                    plsc.addupdate(sb.at[i, sl], rb[i, sl])

        def issue(w):
            p = w % 2
            hs = pltpu.async_copy(
                sp_hbm.at[sidx.at[pl.ds(w * _WIN, _WIN)]], spb[p], ssem[p])
            hr = pltpu.async_copy(
                rp_hbm.at[ridx.at[pl.ds(w * _WIN, _WIN)]], rpb[p], rsem[p])
            g_hand[p] = (hs, hr)

        issue(0)
        for w in range(nwin):
            p = w % 2
            q = 1 - p
            if w + 1 < nwin:
                if o_hand[q] is not None:
                    o_hand[q].wait()
                issue(w + 1)
            hs, hr = g_hand[p]
            hs.wait()
            hr.wait()
            add_window(spb[p], rpb[p])
            o_hand[p] = pltpu.async_copy(
                spb[p], o_hbm.at[pl.ds(e0 + w * _WIN, _WIN)], osem[p])
        for h in o_hand:
            if h is not None:
                h.wait()

    return sc_kernel(sp, rp, senders2d, receivers2d)


_N_CHUNKS = 4


def kernel(edge_features, node_features, senders, receivers,
           W_edge, W_sender, W_receiver, bias):
    n_edges = edge_features.shape[0]
    senders2d = senders.astype(jnp.int32).reshape(1, -1)
    receivers2d = receivers.astype(jnp.int32).reshape(1, -1)
    sp, rp = _node_proj(node_features, W_sender, W_receiver, bias.reshape(1, -1))
    sz = n_edges // _N_CHUNKS
    out = None
    for c in range(_N_CHUNKS):
        sl = slice(c * sz, (c + 1) * sz)
        g_c = _gather_add(sp, rp, senders2d[:, sl], receivers2d[:, sl])
        out = _edge_combine_chunk(out, edge_features[sl], W_edge, g_c, c, n_edges)
    return out
